# Initial kernel scaffold; baseline (speedup 1.0000x reference)
import jax
import jax.numpy as jnp
from jax.experimental import pallas as pl


def kernel(x_nchw, w1, w2, gamma, beta):
    raise NotImplementedError("write your pallas kernel here")


# speedup vs baseline: 1.2785x; 1.2785x over previous
import functools

import jax
import jax.numpy as jnp
from jax.experimental import pallas as pl
from jax.experimental.pallas import tpu as pltpu

_LEAKY_SLOPE = 0.01
_BN_EPS = 1e-5


def _stats_pass(x1_ref, x2_ref, w1_ref, w2_ref, st_ref, *, nb):
    c_half = w1_ref.shape[0]
    w1f = w1_ref[...]
    w2f = w2_ref[...]
    s1 = jnp.zeros((c_half, 1), jnp.float32)
    q1 = jnp.zeros((c_half, 1), jnp.float32)
    s2 = jnp.zeros((c_half, 1), jnp.float32)
    q2 = jnp.zeros((c_half, 1), jnp.float32)
    for i in range(nb):
        y1 = jnp.dot(w1f, x1_ref[i].astype(jnp.float32),
                     preferred_element_type=jnp.float32)
        y2 = jnp.dot(w2f, x2_ref[i].astype(jnp.float32),
                     preferred_element_type=jnp.float32)
        s1 = s1 + jnp.sum(y1, axis=1, keepdims=True)
        q1 = q1 + jnp.sum(y1 * y1, axis=1, keepdims=True)
        s2 = s2 + jnp.sum(y2, axis=1, keepdims=True)
        q2 = q2 + jnp.sum(y2 * y2, axis=1, keepdims=True)
    st_ref[0, :c_half, 0:1] = s1
    st_ref[0, c_half:, 0:1] = s2
    st_ref[0, :c_half, 1:2] = q1
    st_ref[0, c_half:, 1:2] = q2


def _apply_pass(x1_ref, x2_ref, w1_ref, w2_ref, st_ref, g_ref, b_ref, o_ref,
                *, nb, cnt):
    c_half = w1_ref.shape[0]
    st = jnp.sum(st_ref[...], axis=0)
    mean = st[:, 0:1] * (1.0 / cnt)
    var = jnp.maximum(st[:, 1:2] * (1.0 / cnt) - mean * mean, 0.0)
    scale = g_ref[...] * jax.lax.rsqrt(var + _BN_EPS)
    bias = b_ref[...] - mean * scale
    w1f = w1_ref[...]
    w2f = w2_ref[...]
    sc1, sc2 = scale[:c_half], scale[c_half:]
    b1, b2 = bias[:c_half], bias[c_half:]
    for i in range(nb):
        y1 = jnp.dot(w1f, x1_ref[i].astype(jnp.float32),
                     preferred_element_type=jnp.float32)
        y2 = jnp.dot(w2f, x2_ref[i].astype(jnp.float32),
                     preferred_element_type=jnp.float32)
        o_ref[i, :c_half] = y1 * sc1 + b1
        o_ref[i, c_half:] = y2 * sc2 + b2


@jax.jit
def _factorized_reduce(x_nchw, w1, w2, gamma, beta):
    N, C_in, H, W = x_nchw.shape
    C_half = w1.shape[0]
    C_out = 2 * C_half
    Ho, Wo = H // 2, W // 2
    SP = Ho * Wo
    f32 = jnp.float32

    xv = x_nchw.reshape(N, C_in, Ho, 2, Wo, 2)
    xa = jnp.where(xv >= 0, xv, _LEAKY_SLOPE * xv).astype(jnp.bfloat16)
    x1 = xa[:, :, :, 0, :, 0].reshape(N, C_in, SP)
    x2 = xa[:, :, :, 1, :, 1].reshape(N, C_in, SP)

    w1m = w1.reshape(C_half, C_in)
    w2m = w2.reshape(C_half, C_in)
    g_col = gamma.reshape(C_out, 1).astype(f32)
    b_col = beta.reshape(C_out, 1).astype(f32)

    nb = 8 if N % 8 == 0 else 1
    G = N // nb

    stats = pl.pallas_call(
        functools.partial(_stats_pass, nb=nb),
        out_shape=jax.ShapeDtypeStruct((G, C_out, 2), f32),
        grid=(G,),
        in_specs=[
            pl.BlockSpec((nb, C_in, SP), lambda b: (b, 0, 0)),
            pl.BlockSpec((nb, C_in, SP), lambda b: (b, 0, 0)),
            pl.BlockSpec((C_half, C_in), lambda b: (0, 0)),
            pl.BlockSpec((C_half, C_in), lambda b: (0, 0)),
        ],
        out_specs=pl.BlockSpec((1, C_out, 2), lambda b: (b, 0, 0)),
        compiler_params=pltpu.CompilerParams(
            dimension_semantics=("parallel",)),
    )(x1, x2, w1m, w2m)

    out = pl.pallas_call(
        functools.partial(_apply_pass, nb=nb, cnt=float(N * SP)),
        out_shape=jax.ShapeDtypeStruct((N, C_out, SP), f32),
        grid=(G,),
        in_specs=[
            pl.BlockSpec((nb, C_in, SP), lambda b: (b, 0, 0)),
            pl.BlockSpec((nb, C_in, SP), lambda b: (b, 0, 0)),
            pl.BlockSpec((C_half, C_in), lambda b: (0, 0)),
            pl.BlockSpec((C_half, C_in), lambda b: (0, 0)),
            pl.BlockSpec((G, C_out, 2), lambda b: (0, 0, 0)),
            pl.BlockSpec((C_out, 1), lambda b: (0, 0)),
            pl.BlockSpec((C_out, 1), lambda b: (0, 0)),
        ],
        out_specs=pl.BlockSpec((nb, C_out, SP), lambda b: (b, 0, 0)),
        compiler_params=pltpu.CompilerParams(
            dimension_semantics=("parallel",)),
    )(x1, x2, w1m, w2m, stats, g_col, b_col)

    return out.reshape(N, C_out, Ho, Wo)


def kernel(x_nchw, w1, w2, gamma, beta):
    return _factorized_reduce(x_nchw, w1, w2, gamma, beta)


# speedup vs baseline: 1.7484x; 1.3675x over previous
import functools

import jax
import jax.numpy as jnp
from jax.experimental import pallas as pl
from jax.experimental.pallas import tpu as pltpu

_LEAKY_SLOPE = 0.01
_BN_EPS = 1e-5


def _conv_stats_pass(x_ref, w1_ref, w2_ref, p0_ref, p1_ref, y_ref, st_ref,
                     *, nb):
    c_half = w1_ref.shape[0]
    ws = jnp.concatenate([w1_ref[...], w2_ref[...]], axis=0).astype(jnp.bfloat16)
    p0 = p0_ref[...]
    p1 = p1_ref[...]
    s1 = jnp.zeros((c_half, 1), jnp.float32)
    q1 = jnp.zeros((c_half, 1), jnp.float32)
    s2 = jnp.zeros((c_half, 1), jnp.float32)
    q2 = jnp.zeros((c_half, 1), jnp.float32)
    for i in range(nb):
        xb = x_ref[i]
        xb = jnp.where(xb >= 0, xb, _LEAKY_SLOPE * xb).astype(jnp.bfloat16)
        a = jnp.dot(ws, xb, preferred_element_type=jnp.float32)
        ab = a.astype(jnp.bfloat16)
        y1 = jnp.dot(ab[:c_half], p0, preferred_element_type=jnp.float32)
        y2 = jnp.dot(ab[c_half:], p1, preferred_element_type=jnp.float32)
        s1 = s1 + jnp.sum(y1, axis=1, keepdims=True)
        q1 = q1 + jnp.sum(y1 * y1, axis=1, keepdims=True)
        s2 = s2 + jnp.sum(y2, axis=1, keepdims=True)
        q2 = q2 + jnp.sum(y2 * y2, axis=1, keepdims=True)
        y_ref[i, :c_half] = y1.astype(jnp.bfloat16)
        y_ref[i, c_half:] = y2.astype(jnp.bfloat16)
    st_ref[0, :c_half, 0:1] = s1
    st_ref[0, c_half:, 0:1] = s2
    st_ref[0, :c_half, 1:2] = q1
    st_ref[0, c_half:, 1:2] = q2


def _bn_apply_pass(y_ref, st_ref, g_ref, b_ref, o_ref, *, cnt):
    st = jnp.sum(st_ref[...], axis=0)
    mean = st[:, 0:1] * (1.0 / cnt)
    var = jnp.maximum(st[:, 1:2] * (1.0 / cnt) - mean * mean, 0.0)
    scale = g_ref[...] * jax.lax.rsqrt(var + _BN_EPS)
    bias = b_ref[...] - mean * scale
    y = y_ref[...].astype(jnp.float32)
    o_ref[...] = y * scale[None] + bias[None]


@jax.jit
def _factorized_reduce(x_nchw, w1, w2, gamma, beta):
    N, C_in, H, W = x_nchw.shape
    C_half = w1.shape[0]
    C_out = 2 * C_half
    Ho, Wo = H // 2, W // 2
    SP = Ho * Wo
    HW = H * W
    f32 = jnp.float32

    x_flat = x_nchw.reshape(N, C_in, HW)
    w1m = w1.reshape(C_half, C_in)
    w2m = w2.reshape(C_half, C_in)
    g_col = gamma.reshape(C_out, 1).astype(f32)
    b_col = beta.reshape(C_out, 1).astype(f32)

    lanes = jnp.arange(HW, dtype=jnp.int32)[:, None]
    q = jnp.arange(SP, dtype=jnp.int32)[None, :]
    src0 = (2 * (q // Wo)) * W + 2 * (q % Wo)
    p0 = (lanes == src0).astype(jnp.bfloat16)
    p1 = (lanes == src0 + W + 1).astype(jnp.bfloat16)

    nb = 8 if N % 8 == 0 else 1
    G = N // nb

    y, stats = pl.pallas_call(
        functools.partial(_conv_stats_pass, nb=nb),
        out_shape=(jax.ShapeDtypeStruct((N, C_out, SP), jnp.bfloat16),
                   jax.ShapeDtypeStruct((G, C_out, 2), f32)),
        grid=(G,),
        in_specs=[
            pl.BlockSpec((nb, C_in, HW), lambda b: (b, 0, 0)),
            pl.BlockSpec((C_half, C_in), lambda b: (0, 0)),
            pl.BlockSpec((C_half, C_in), lambda b: (0, 0)),
            pl.BlockSpec((HW, SP), lambda b: (0, 0)),
            pl.BlockSpec((HW, SP), lambda b: (0, 0)),
        ],
        out_specs=(pl.BlockSpec((nb, C_out, SP), lambda b: (b, 0, 0)),
                   pl.BlockSpec((1, C_out, 2), lambda b: (b, 0, 0))),
        compiler_params=pltpu.CompilerParams(
            dimension_semantics=("parallel",)),
    )(x_flat, w1m, w2m, p0, p1)

    out = pl.pallas_call(
        functools.partial(_bn_apply_pass, cnt=float(N * SP)),
        out_shape=jax.ShapeDtypeStruct((N, C_out, SP), f32),
        grid=(G,),
        in_specs=[
            pl.BlockSpec((nb, C_out, SP), lambda b: (b, 0, 0)),
            pl.BlockSpec((G, C_out, 2), lambda b: (0, 0, 0)),
            pl.BlockSpec((C_out, 1), lambda b: (0, 0)),
            pl.BlockSpec((C_out, 1), lambda b: (0, 0)),
        ],
        out_specs=pl.BlockSpec((nb, C_out, SP), lambda b: (b, 0, 0)),
        compiler_params=pltpu.CompilerParams(
            dimension_semantics=("parallel",)),
    )(y, stats, g_col, b_col)

    return out.reshape(N, C_out, Ho, Wo)


def kernel(x_nchw, w1, w2, gamma, beta):
    return _factorized_reduce(x_nchw, w1, w2, gamma, beta)


# speedup vs baseline: 1.8181x; 1.0399x over previous
import functools

import jax
import jax.numpy as jnp
from jax.experimental import pallas as pl
from jax.experimental.pallas import tpu as pltpu

_LEAKY_SLOPE = 0.01
_BN_EPS = 1e-5


def _fused_kernel(x_ref, w1_ref, w2_ref, p0_ref, p1_ref, g_ref, bb_ref,
                  o_ref, y_scr, st_scr, *, nb, cnt):
    p = pl.program_id(0)
    b = pl.program_id(1)
    c_half = w1_ref.shape[0]

    @pl.when(jnp.logical_and(p == 0, b == 0))
    def _init():
        st_scr[...] = jnp.zeros_like(st_scr)

    @pl.when(p == 0)
    def _conv_stats():
        ws = jnp.concatenate([w1_ref[...], w2_ref[...]],
                             axis=0).astype(jnp.bfloat16)
        p0 = p0_ref[...]
        p1 = p1_ref[...]
        s1 = jnp.zeros((c_half, 1), jnp.float32)
        q1 = jnp.zeros((c_half, 1), jnp.float32)
        s2 = jnp.zeros((c_half, 1), jnp.float32)
        q2 = jnp.zeros((c_half, 1), jnp.float32)
        for i in range(nb):
            xb = x_ref[i]
            xb = jnp.where(xb >= 0, xb, _LEAKY_SLOPE * xb).astype(jnp.bfloat16)
            a = jnp.dot(ws, xb, preferred_element_type=jnp.float32)
            ab = a.astype(jnp.bfloat16)
            y1 = jnp.dot(ab[:c_half], p0, preferred_element_type=jnp.float32)
            y2 = jnp.dot(ab[c_half:], p1, preferred_element_type=jnp.float32)
            s1 = s1 + jnp.sum(y1, axis=1, keepdims=True)
            q1 = q1 + jnp.sum(y1 * y1, axis=1, keepdims=True)
            s2 = s2 + jnp.sum(y2, axis=1, keepdims=True)
            q2 = q2 + jnp.sum(y2 * y2, axis=1, keepdims=True)
            y_scr[b * nb + i, :c_half] = y1.astype(jnp.bfloat16)
            y_scr[b * nb + i, c_half:] = y2.astype(jnp.bfloat16)
        st_scr[:c_half, 0:1] += s1
        st_scr[c_half:, 0:1] += s2
        st_scr[:c_half, 1:2] += q1
        st_scr[c_half:, 1:2] += q2

    @pl.when(p == 1)
    def _bn_apply():
        st = st_scr[...]
        mean = st[:, 0:1] * (1.0 / cnt)
        var = jnp.maximum(st[:, 1:2] * (1.0 / cnt) - mean * mean, 0.0)
        scale = g_ref[...] * jax.lax.rsqrt(var + _BN_EPS)
        bias = bb_ref[...] - mean * scale
        y = y_scr[pl.ds(b * nb, nb)].astype(jnp.float32)
        o_ref[...] = y * scale[None] + bias[None]


@jax.jit
def _factorized_reduce(x_nchw, w1, w2, gamma, beta):
    N, C_in, H, W = x_nchw.shape
    C_half = w1.shape[0]
    C_out = 2 * C_half
    Ho, Wo = H // 2, W // 2
    SP = Ho * Wo
    HW = H * W
    f32 = jnp.float32

    x_flat = x_nchw.reshape(N, C_in, HW)
    w1m = w1.reshape(C_half, C_in)
    w2m = w2.reshape(C_half, C_in)
    g_col = gamma.reshape(C_out, 1).astype(f32)
    b_col = beta.reshape(C_out, 1).astype(f32)

    lanes = jnp.arange(HW, dtype=jnp.int32)[:, None]
    q = jnp.arange(SP, dtype=jnp.int32)[None, :]
    src0 = (2 * (q // Wo)) * W + 2 * (q % Wo)
    p0 = (lanes == src0).astype(jnp.bfloat16)
    p1 = (lanes == src0 + W + 1).astype(jnp.bfloat16)

    nb = 8 if N % 8 == 0 else 1
    G = N // nb

    out = pl.pallas_call(
        functools.partial(_fused_kernel, nb=nb, cnt=float(N * SP)),
        out_shape=jax.ShapeDtypeStruct((N, C_out, SP), f32),
        grid=(2, G),
        in_specs=[
            pl.BlockSpec((nb, C_in, HW),
                         lambda p, b: (b * (1 - p) + (G - 1) * p, 0, 0)),
            pl.BlockSpec((C_half, C_in), lambda p, b: (0, 0)),
            pl.BlockSpec((C_half, C_in), lambda p, b: (0, 0)),
            pl.BlockSpec((HW, SP), lambda p, b: (0, 0)),
            pl.BlockSpec((HW, SP), lambda p, b: (0, 0)),
            pl.BlockSpec((C_out, 1), lambda p, b: (0, 0)),
            pl.BlockSpec((C_out, 1), lambda p, b: (0, 0)),
        ],
        out_specs=pl.BlockSpec((nb, C_out, SP), lambda p, b: (b * p, 0, 0)),
        scratch_shapes=[
            pltpu.VMEM((N, C_out, SP), jnp.bfloat16),
            pltpu.VMEM((C_out, 2), f32),
        ],
        compiler_params=pltpu.CompilerParams(
            dimension_semantics=("arbitrary", "arbitrary")),
    )(x_flat, w1m, w2m, p0, p1, g_col, b_col)

    return out.reshape(N, C_out, Ho, Wo)


def kernel(x_nchw, w1, w2, gamma, beta):
    return _factorized_reduce(x_nchw, w1, w2, gamma, beta)


# speedup vs baseline: 1.8929x; 1.0411x over previous
import functools

import jax
import jax.numpy as jnp
from jax.experimental import pallas as pl
from jax.experimental.pallas import tpu as pltpu

_LEAKY_SLOPE = 0.01
_BN_EPS = 1e-5


def _fused_kernel(x_ref, w1_ref, w2_ref, p0_ref, p1_ref, g_ref, bb_ref,
                  o_ref, y_scr, st_scr, *, nb, cnt):
    p = pl.program_id(0)
    b = pl.program_id(1)
    c_half = w1_ref.shape[0]

    @pl.when(jnp.logical_and(p == 0, b == 0))
    def _init():
        st_scr[...] = jnp.zeros_like(st_scr)

    @pl.when(p == 0)
    def _conv_stats():
        ws = jnp.concatenate([w1_ref[...], w2_ref[...]], axis=0)
        p0 = p0_ref[...]
        p1 = p1_ref[...]
        s1 = jnp.zeros((c_half, 1), jnp.float32)
        q1 = jnp.zeros((c_half, 1), jnp.float32)
        s2 = jnp.zeros((c_half, 1), jnp.float32)
        q2 = jnp.zeros((c_half, 1), jnp.float32)
        for i in range(nb):
            xb = x_ref[i]
            xb = jnp.where(xb >= 0, xb, _LEAKY_SLOPE * xb)
            a = jnp.dot(ws, xb, preferred_element_type=jnp.float32)
            y1 = jnp.dot(a[:c_half], p0, preferred_element_type=jnp.float32)
            y2 = jnp.dot(a[c_half:], p1, preferred_element_type=jnp.float32)
            s1 = s1 + jnp.sum(y1, axis=1, keepdims=True)
            q1 = q1 + jnp.sum(y1 * y1, axis=1, keepdims=True)
            s2 = s2 + jnp.sum(y2, axis=1, keepdims=True)
            q2 = q2 + jnp.sum(y2 * y2, axis=1, keepdims=True)
            y_scr[b * nb + i, :c_half] = y1
            y_scr[b * nb + i, c_half:] = y2
        st_scr[:c_half, 0:1] += s1
        st_scr[c_half:, 0:1] += s2
        st_scr[:c_half, 1:2] += q1
        st_scr[c_half:, 1:2] += q2

    @pl.when(p == 1)
    def _bn_apply():
        st = st_scr[...]
        mean = st[:, 0:1] * (1.0 / cnt)
        var = jnp.maximum(st[:, 1:2] * (1.0 / cnt) - mean * mean, 0.0)
        scale = g_ref[...] * jax.lax.rsqrt(var + _BN_EPS)
        bias = bb_ref[...] - mean * scale
        y = y_scr[pl.ds(b * nb, nb)]
        o_ref[...] = y * scale[None] + bias[None]


@jax.jit
def _factorized_reduce(x_nchw, w1, w2, gamma, beta):
    N, C_in, H, W = x_nchw.shape
    C_half = w1.shape[0]
    C_out = 2 * C_half
    Ho, Wo = H // 2, W // 2
    SP = Ho * Wo
    HW = H * W
    f32 = jnp.float32

    x_flat = x_nchw.reshape(N, C_in, HW)
    w1m = w1.reshape(C_half, C_in)
    w2m = w2.reshape(C_half, C_in)
    g_col = gamma.reshape(C_out, 1).astype(f32)
    b_col = beta.reshape(C_out, 1).astype(f32)

    lanes = jnp.arange(HW, dtype=jnp.int32)[:, None]
    q = jnp.arange(SP, dtype=jnp.int32)[None, :]
    src0 = (2 * (q // Wo)) * W + 2 * (q % Wo)
    p0 = (lanes == src0).astype(f32)
    p1 = (lanes == src0 + W + 1).astype(f32)

    nb = 8 if N % 8 == 0 else 1
    G = N // nb

    out = pl.pallas_call(
        functools.partial(_fused_kernel, nb=nb, cnt=float(N * SP)),
        out_shape=jax.ShapeDtypeStruct((N, C_out, SP), f32),
        grid=(2, G),
        in_specs=[
            pl.BlockSpec((nb, C_in, HW),
                         lambda p, b: (b * (1 - p) + (G - 1) * p, 0, 0)),
            pl.BlockSpec((C_half, C_in), lambda p, b: (0, 0)),
            pl.BlockSpec((C_half, C_in), lambda p, b: (0, 0)),
            pl.BlockSpec((HW, SP), lambda p, b: (0, 0)),
            pl.BlockSpec((HW, SP), lambda p, b: (0, 0)),
            pl.BlockSpec((C_out, 1), lambda p, b: (0, 0)),
            pl.BlockSpec((C_out, 1), lambda p, b: (0, 0)),
        ],
        out_specs=pl.BlockSpec((nb, C_out, SP), lambda p, b: (b * p, 0, 0)),
        scratch_shapes=[
            pltpu.VMEM((N, C_out, SP), f32),
            pltpu.VMEM((C_out, 2), f32),
        ],
        compiler_params=pltpu.CompilerParams(
            dimension_semantics=("arbitrary", "arbitrary"),
            vmem_limit_bytes=42 * 1024 * 1024),
    )(x_flat, w1m, w2m, p0, p1, g_col, b_col)

    return out.reshape(N, C_out, Ho, Wo)


def kernel(x_nchw, w1, w2, gamma, beta):
    return _factorized_reduce(x_nchw, w1, w2, gamma, beta)


# speedup vs baseline: 1.9011x; 1.0044x over previous
import functools

import jax
import jax.numpy as jnp
from jax.experimental import pallas as pl
from jax.experimental.pallas import tpu as pltpu

_LEAKY_SLOPE = 0.01
_BN_EPS = 1e-5


def _fused_kernel(x_ref, w1_ref, w2_ref, p0_ref, p1_ref, g_ref, bb_ref,
                  o_ref, y_scr, st_scr, *, nb, cnt):
    p = pl.program_id(0)
    b = pl.program_id(1)
    c_half = w1_ref.shape[0]

    @pl.when(jnp.logical_and(p == 0, b == 0))
    def _init():
        st_scr[...] = jnp.zeros_like(st_scr)

    @pl.when(p == 0)
    def _conv_stats():
        ws = jnp.concatenate([w1_ref[...], w2_ref[...]], axis=0)
        p0 = p0_ref[...]
        p1 = p1_ref[...]
        s1 = jnp.zeros((c_half, 1), jnp.float32)
        q1 = jnp.zeros((c_half, 1), jnp.float32)
        s2 = jnp.zeros((c_half, 1), jnp.float32)
        q2 = jnp.zeros((c_half, 1), jnp.float32)
        for i in range(nb):
            xb = x_ref[i]
            xb = jnp.where(xb >= 0, xb, _LEAKY_SLOPE * xb)
            a = jnp.dot(ws, xb, preferred_element_type=jnp.float32)
            y1 = jnp.dot(a[:c_half], p0, preferred_element_type=jnp.float32)
            y2 = jnp.dot(a[c_half:], p1, preferred_element_type=jnp.float32)
            s1 = s1 + jnp.sum(y1, axis=1, keepdims=True)
            q1 = q1 + jnp.sum(y1 * y1, axis=1, keepdims=True)
            s2 = s2 + jnp.sum(y2, axis=1, keepdims=True)
            q2 = q2 + jnp.sum(y2 * y2, axis=1, keepdims=True)
            y_scr[b * nb + i, :c_half] = y1
            y_scr[b * nb + i, c_half:] = y2
        st_scr[:c_half, 0:1] += s1
        st_scr[c_half:, 0:1] += s2
        st_scr[:c_half, 1:2] += q1
        st_scr[c_half:, 1:2] += q2

    @pl.when(p == 1)
    def _bn_apply():
        st = st_scr[...]
        mean = st[:, 0:1] * (1.0 / cnt)
        var = jnp.maximum(st[:, 1:2] * (1.0 / cnt) - mean * mean, 0.0)
        scale = g_ref[...] * jax.lax.rsqrt(var + _BN_EPS)
        bias = bb_ref[...] - mean * scale
        y = y_scr[pl.ds(b * nb, nb)]
        o_ref[...] = y * scale[None] + bias[None]


@jax.jit
def _factorized_reduce(x_nchw, w1, w2, gamma, beta):
    N, C_in, H, W = x_nchw.shape
    C_half = w1.shape[0]
    C_out = 2 * C_half
    Ho, Wo = H // 2, W // 2
    SP = Ho * Wo
    HW = H * W
    f32 = jnp.float32

    x_flat = x_nchw.reshape(N, C_in, HW)
    w1m = w1.reshape(C_half, C_in)
    w2m = w2.reshape(C_half, C_in)
    g_col = gamma.reshape(C_out, 1).astype(f32)
    b_col = beta.reshape(C_out, 1).astype(f32)

    lanes = jnp.arange(HW, dtype=jnp.int32)[:, None]
    q = jnp.arange(SP, dtype=jnp.int32)[None, :]
    src0 = (2 * (q // Wo)) * W + 2 * (q % Wo)
    p0 = (lanes == src0).astype(f32)
    p1 = (lanes == src0 + W + 1).astype(f32)

    nb = 16 if N % 16 == 0 else (8 if N % 8 == 0 else 1)
    G = N // nb

    out = pl.pallas_call(
        functools.partial(_fused_kernel, nb=nb, cnt=float(N * SP)),
        out_shape=jax.ShapeDtypeStruct((N, C_out, SP), f32),
        grid=(2, G),
        in_specs=[
            pl.BlockSpec((nb, C_in, HW),
                         lambda p, b: (b * (1 - p) + (G - 1) * p, 0, 0)),
            pl.BlockSpec((C_half, C_in), lambda p, b: (0, 0)),
            pl.BlockSpec((C_half, C_in), lambda p, b: (0, 0)),
            pl.BlockSpec((HW, SP), lambda p, b: (0, 0)),
            pl.BlockSpec((HW, SP), lambda p, b: (0, 0)),
            pl.BlockSpec((C_out, 1), lambda p, b: (0, 0)),
            pl.BlockSpec((C_out, 1), lambda p, b: (0, 0)),
        ],
        out_specs=pl.BlockSpec((nb, C_out, SP), lambda p, b: (b * p, 0, 0)),
        scratch_shapes=[
            pltpu.VMEM((N, C_out, SP), f32),
            pltpu.VMEM((C_out, 2), f32),
        ],
        compiler_params=pltpu.CompilerParams(
            dimension_semantics=("arbitrary", "arbitrary"),
            vmem_limit_bytes=42 * 1024 * 1024),
    )(x_flat, w1m, w2m, p0, p1, g_col, b_col)

    return out.reshape(N, C_out, Ho, Wo)


def kernel(x_nchw, w1, w2, gamma, beta):
    return _factorized_reduce(x_nchw, w1, w2, gamma, beta)


# speedup vs baseline: 1.9120x; 1.0057x over previous
import functools

import jax
import jax.numpy as jnp
from jax.experimental import pallas as pl
from jax.experimental.pallas import tpu as pltpu

_LEAKY_SLOPE = 0.01
_BN_EPS = 1e-5


def _fused_kernel(x_ref, w1_ref, w2_ref, p0_ref, p1_ref, g_ref, bb_ref,
                  o_ref, y_scr, st_scr, *, nb, cnt):
    p = pl.program_id(0)
    b = pl.program_id(1)
    c_half = w1_ref.shape[0]

    @pl.when(jnp.logical_and(p == 0, b == 0))
    def _init():
        st_scr[...] = jnp.zeros_like(st_scr)

    @pl.when(p == 0)
    def _conv_stats():
        ws = jnp.concatenate([w1_ref[...], w2_ref[...]], axis=0)
        p0 = p0_ref[...]
        p1 = p1_ref[...]
        s1 = jnp.zeros((c_half, 1), jnp.float32)
        q1 = jnp.zeros((c_half, 1), jnp.float32)
        s2 = jnp.zeros((c_half, 1), jnp.float32)
        q2 = jnp.zeros((c_half, 1), jnp.float32)
        for i in range(nb):
            xb = x_ref[i]
            xb = jnp.maximum(xb, _LEAKY_SLOPE * xb)
            a = jnp.dot(ws, xb, preferred_element_type=jnp.float32)
            y1 = jnp.dot(a[:c_half], p0, preferred_element_type=jnp.float32)
            y2 = jnp.dot(a[c_half:], p1, preferred_element_type=jnp.float32)
            s1 = s1 + jnp.sum(y1, axis=1, keepdims=True)
            q1 = q1 + jnp.sum(y1 * y1, axis=1, keepdims=True)
            s2 = s2 + jnp.sum(y2, axis=1, keepdims=True)
            q2 = q2 + jnp.sum(y2 * y2, axis=1, keepdims=True)
            y_scr[b * nb + i, :c_half] = y1
            y_scr[b * nb + i, c_half:] = y2
        st_scr[:c_half, 0:1] += s1
        st_scr[c_half:, 0:1] += s2
        st_scr[:c_half, 1:2] += q1
        st_scr[c_half:, 1:2] += q2

    @pl.when(p == 1)
    def _bn_apply():
        st = st_scr[...]
        mean = st[:, 0:1] * (1.0 / cnt)
        var = jnp.maximum(st[:, 1:2] * (1.0 / cnt) - mean * mean, 0.0)
        scale = g_ref[...] * jax.lax.rsqrt(var + _BN_EPS)
        bias = bb_ref[...] - mean * scale
        y = y_scr[pl.ds(b * nb, nb)]
        o_ref[...] = y * scale[None] + bias[None]


@jax.jit
def _factorized_reduce(x_nchw, w1, w2, gamma, beta):
    N, C_in, H, W = x_nchw.shape
    C_half = w1.shape[0]
    C_out = 2 * C_half
    Ho, Wo = H // 2, W // 2
    SP = Ho * Wo
    HW = H * W
    f32 = jnp.float32

    x_flat = x_nchw.reshape(N, C_in, HW)
    w1m = w1.reshape(C_half, C_in)
    w2m = w2.reshape(C_half, C_in)
    g_col = gamma.reshape(C_out, 1).astype(f32)
    b_col = beta.reshape(C_out, 1).astype(f32)

    lanes = jnp.arange(HW, dtype=jnp.int32)[:, None]
    q = jnp.arange(SP, dtype=jnp.int32)[None, :]
    src0 = (2 * (q // Wo)) * W + 2 * (q % Wo)
    p0 = (lanes == src0).astype(f32)
    p1 = (lanes == src0 + W + 1).astype(f32)

    nb = 16 if N % 16 == 0 else (8 if N % 8 == 0 else 1)
    G = N // nb

    out = pl.pallas_call(
        functools.partial(_fused_kernel, nb=nb, cnt=float(N * SP)),
        out_shape=jax.ShapeDtypeStruct((N, C_out, SP), f32),
        grid=(2, G),
        in_specs=[
            pl.BlockSpec((nb, C_in, HW),
                         lambda p, b: (b * (1 - p) + (G - 1) * p, 0, 0)),
            pl.BlockSpec((C_half, C_in), lambda p, b: (0, 0)),
            pl.BlockSpec((C_half, C_in), lambda p, b: (0, 0)),
            pl.BlockSpec((HW, SP), lambda p, b: (0, 0)),
            pl.BlockSpec((HW, SP), lambda p, b: (0, 0)),
            pl.BlockSpec((C_out, 1), lambda p, b: (0, 0)),
            pl.BlockSpec((C_out, 1), lambda p, b: (0, 0)),
        ],
        out_specs=pl.BlockSpec((nb, C_out, SP), lambda p, b: (b * p, 0, 0)),
        scratch_shapes=[
            pltpu.VMEM((N, C_out, SP), f32),
            pltpu.VMEM((C_out, 2), f32),
        ],
        compiler_params=pltpu.CompilerParams(
            dimension_semantics=("arbitrary", "arbitrary"),
            vmem_limit_bytes=42 * 1024 * 1024),
    )(x_flat, w1m, w2m, p0, p1, g_col, b_col)

    return out.reshape(N, C_out, Ho, Wo)


def kernel(x_nchw, w1, w2, gamma, beta):
    return _factorized_reduce(x_nchw, w1, w2, gamma, beta)


# speedup vs baseline: 1.9274x; 1.0081x over previous
import functools

import jax
import jax.numpy as jnp
from jax.experimental import pallas as pl
from jax.experimental.pallas import tpu as pltpu

_LEAKY_SLOPE = 0.01
_BN_EPS = 1e-5


def _fused_kernel(x_ref, w1_ref, w2_ref, p0_ref, p1_ref, g_ref, bb_ref,
                  o_ref, y_scr, st_scr, *, nb, cnt):
    p = pl.program_id(0)
    b = pl.program_id(1)
    c_half = w1_ref.shape[0]

    @pl.when(p == 0)
    def _conv():
        ws = jnp.concatenate([w1_ref[...], w2_ref[...]], axis=0)
        p0 = p0_ref[...]
        p1 = p1_ref[...]
        for i in range(nb):
            xb = x_ref[i]
            xb = jnp.maximum(xb, _LEAKY_SLOPE * xb)
            a = jnp.dot(ws, xb, preferred_element_type=jnp.float32)
            y1 = jnp.dot(a[:c_half], p0, preferred_element_type=jnp.float32)
            y2 = jnp.dot(a[c_half:], p1, preferred_element_type=jnp.float32)
            y_scr[b * nb + i, :c_half] = y1
            y_scr[b * nb + i, c_half:] = y2

    @pl.when(jnp.logical_and(p == 1, b == 0))
    def _fold_stats():
        n_img = y_scr.shape[0]
        c_out = y_scr.shape[1]
        ch = 8
        s = jnp.zeros((c_out, 1), jnp.float32)
        q = jnp.zeros((c_out, 1), jnp.float32)
        for g in range(n_img // ch):
            t = y_scr[pl.ds(g * ch, ch)]
            ts = jnp.sum(t, axis=0)
            tq = jnp.sum(t * t, axis=0)
            s = s + jnp.sum(ts, axis=1, keepdims=True)
            q = q + jnp.sum(tq, axis=1, keepdims=True)
        mean = s * (1.0 / cnt)
        var = jnp.maximum(q * (1.0 / cnt) - mean * mean, 0.0)
        scale = g_ref[...] * jax.lax.rsqrt(var + _BN_EPS)
        bias = bb_ref[...] - mean * scale
        st_scr[:, 0:1] = scale
        st_scr[:, 1:2] = bias

    @pl.when(p == 1)
    def _bn_apply():
        scale = st_scr[:, 0:1]
        bias = st_scr[:, 1:2]
        y = y_scr[pl.ds(b * nb, nb)]
        o_ref[...] = y * scale[None] + bias[None]


@jax.jit
def _factorized_reduce(x_nchw, w1, w2, gamma, beta):
    N, C_in, H, W = x_nchw.shape
    C_half = w1.shape[0]
    C_out = 2 * C_half
    Ho, Wo = H // 2, W // 2
    SP = Ho * Wo
    HW = H * W
    f32 = jnp.float32

    x_flat = x_nchw.reshape(N, C_in, HW)
    w1m = w1.reshape(C_half, C_in)
    w2m = w2.reshape(C_half, C_in)
    g_col = gamma.reshape(C_out, 1).astype(f32)
    b_col = beta.reshape(C_out, 1).astype(f32)

    lanes = jnp.arange(HW, dtype=jnp.int32)[:, None]
    q = jnp.arange(SP, dtype=jnp.int32)[None, :]
    src0 = (2 * (q // Wo)) * W + 2 * (q % Wo)
    p0 = (lanes == src0).astype(f32)
    p1 = (lanes == src0 + W + 1).astype(f32)

    nb = 16 if N % 16 == 0 else (8 if N % 8 == 0 else 1)
    G = N // nb

    out = pl.pallas_call(
        functools.partial(_fused_kernel, nb=nb, cnt=float(N * SP)),
        out_shape=jax.ShapeDtypeStruct((N, C_out, SP), f32),
        grid=(2, G),
        in_specs=[
            pl.BlockSpec((nb, C_in, HW),
                         lambda p, b: (b * (1 - p) + (G - 1) * p, 0, 0)),
            pl.BlockSpec((C_half, C_in), lambda p, b: (0, 0)),
            pl.BlockSpec((C_half, C_in), lambda p, b: (0, 0)),
            pl.BlockSpec((HW, SP), lambda p, b: (0, 0)),
            pl.BlockSpec((HW, SP), lambda p, b: (0, 0)),
            pl.BlockSpec((C_out, 1), lambda p, b: (0, 0)),
            pl.BlockSpec((C_out, 1), lambda p, b: (0, 0)),
        ],
        out_specs=pl.BlockSpec((nb, C_out, SP), lambda p, b: (b * p, 0, 0)),
        scratch_shapes=[
            pltpu.VMEM((N, C_out, SP), f32),
            pltpu.VMEM((C_out, 2), f32),
        ],
        compiler_params=pltpu.CompilerParams(
            dimension_semantics=("arbitrary", "arbitrary"),
            vmem_limit_bytes=42 * 1024 * 1024),
    )(x_flat, w1m, w2m, p0, p1, g_col, b_col)

    return out.reshape(N, C_out, Ho, Wo)


def kernel(x_nchw, w1, w2, gamma, beta):
    return _factorized_reduce(x_nchw, w1, w2, gamma, beta)
